# baseline (speedup 1.0000x reference)
import functools

import jax
import jax.numpy as jnp
from jax import lax
from jax.experimental import pallas as pl
from jax.experimental.pallas import tpu as pltpu
from jax.experimental.pallas import tpu_sc as plsc

_L = 16
_NCHUNK = 4


def _sc_gather(emb, pp_flat, idx):
    V, D = emb.shape
    V3 = pp_flat.shape[0]
    B = idx.shape[0]
    info = plsc.get_sparse_core_info()
    nw = info.num_subcores
    bw = B // nw
    cs = bw // _NCHUNK
    groups = bw // _L
    mesh = plsc.VectorSubcoreMesh(core_axis_name="c", subcore_axis_name="s",
                                  num_cores=1)

    @functools.partial(
        pl.kernel,
        mesh=mesh,
        compiler_params=pltpu.CompilerParams(
            use_tc_tiling_on_sc=False, needs_layout_passes=False),
        out_type=(jax.ShapeDtypeStruct((B, D), jnp.float32),
                  jax.ShapeDtypeStruct((B, 2), jnp.float32)),
        scratch_types=[
            pltpu.VMEM((bw,), jnp.int32),
            pltpu.VMEM((bw, D), jnp.float32),
            pltpu.VMEM((V3,), jnp.float32),
            pltpu.VMEM((bw, 2), jnp.float32),
            [pltpu.SemaphoreType.DMA] * _NCHUNK,
            [pltpu.SemaphoreType.DMA] * _NCHUNK,
            pltpu.SemaphoreType.DMA,
        ],
    )
    def gather_kernel(emb_hbm, pp_hbm, idx_hbm, gemb_hbm, bv_hbm,
                      idx_v, rows_v, pp_v, bv_v, gsems, osems, psem):
        wid = lax.axis_index("s")
        base = wid * bw
        pltpu.sync_copy(idx_hbm.at[pl.ds(base, bw)], idx_v)
        ppc = pltpu.async_copy(pp_hbm, pp_v, psem)
        gathers = []
        for c in range(_NCHUNK):
            gathers.append(pltpu.async_copy(
                emb_hbm.at[idx_v.at[pl.ds(c * cs, cs)]],
                rows_v.at[pl.ds(c * cs, cs)], gsems[c]))

        ppc.wait()
        iota = lax.broadcasted_iota(jnp.int32, (_L,), 0)
        zc = iota * 0

        @plsc.parallel_loop(0, groups)
        def body(g):
            off = g * _L
            iv = idx_v[pl.ds(off, _L)]
            rows = iota + off
            i3 = iv * 3
            plsc.store_scatter(bv_v, [rows, zc],
                               plsc.load_gather(pp_v, [i3]))
            plsc.store_scatter(bv_v, [rows, zc + 1],
                               plsc.load_gather(pp_v, [i3 + 1]))

        writes = [pltpu.async_copy(bv_v, bv_hbm.at[pl.ds(base, bw)],
                                   osems[0])]
        for c in range(_NCHUNK):
            gathers[c].wait()
            writes.append(pltpu.async_copy(
                rows_v.at[pl.ds(c * cs, cs)],
                gemb_hbm.at[pl.ds(base + c * cs, cs)], osems[c]))
        for w in writes:
            w.wait()

    return gather_kernel(emb, pp_flat, idx)


def _tc_mlp(g, bv, ig, W1, b1, W2, b2, W3, b3):
    B = g.shape[0]
    blk = 8192
    grid = (B // blk,)

    def body(g_ref, bv_ref, ig_ref, w1_ref, b1_ref, w2_ref, b2_ref,
             w3_ref, b3_ref, out_ref):
        emb = g_ref[...]
        w1 = w1_ref[...]
        dn = (((1,), (1,)), ((), ()))
        h = lax.dot_general(emb, w1[:, :8], dn,
                            preferred_element_type=jnp.float32)
        h = h + ig_ref[...] * w1[:, 8][None, :] + b1_ref[...]
        h = jnp.maximum(h, 0.0)
        h = lax.dot_general(h, w2_ref[...], dn,
                            preferred_element_type=jnp.float32)
        h = jnp.maximum(h + b2_ref[...], 0.0)
        res = lax.dot_general(h, w3_ref[...], dn,
                              preferred_element_type=jnp.float32)
        out_ref[...] = bv_ref[...] + res + b3_ref[...]

    full = lambda shape: pl.BlockSpec(shape, lambda i: (0, 0))
    return pl.pallas_call(
        body,
        grid=grid,
        in_specs=[
            pl.BlockSpec((blk, 8), lambda i: (i, 0)),
            pl.BlockSpec((blk, 2), lambda i: (i, 0)),
            pl.BlockSpec((blk, 1), lambda i: (i, 0)),
            full((32, 9)),
            full((1, 32)),
            full((16, 32)),
            full((1, 16)),
            full((2, 16)),
            full((1, 2)),
        ],
        out_specs=pl.BlockSpec((blk, 2), lambda i: (i, 0)),
        out_shape=jax.ShapeDtypeStruct((B, 2), jnp.float32),
    )(g, bv, ig, W1, b1, W2, b2, W3, b3)


def kernel(action_idx, is_ground, physics_params, action_emb,
           W1, b1, W2, b2, W3, b3, gravity):
    B = action_idx.shape[0]
    idx = action_idx.astype(jnp.int32)
    g, bv = _sc_gather(action_emb, physics_params.reshape(-1), idx)
    out = _tc_mlp(g, bv, is_ground.reshape(B, 1), W1, b1.reshape(1, 32),
                  W2, b2.reshape(1, 16), W3, b3.reshape(1, 2))
    return (out, gravity)


# speedup vs baseline: 1.1878x; 1.1878x over previous
import functools

import jax
import jax.numpy as jnp
from jax import lax
from jax.experimental import pallas as pl
from jax.experimental.pallas import tpu as pltpu
from jax.experimental.pallas import tpu_sc as plsc

_TBL_W = 16
_NCHUNK = 4


def _sc_gather(table, idx):
    V, D = table.shape
    B = idx.shape[0]
    info = plsc.get_sparse_core_info()
    nw = 1 * info.num_subcores
    b_per_w = B // nw
    cs = b_per_w // _NCHUNK
    mesh = plsc.VectorSubcoreMesh(core_axis_name="c", subcore_axis_name="s",
                                  num_cores=1)

    @functools.partial(
        pl.kernel,
        mesh=mesh,
        compiler_params=pltpu.CompilerParams(use_tc_tiling_on_sc=False),
        out_type=jax.ShapeDtypeStruct((B, D), jnp.float32),
        scratch_types=[
            pltpu.VMEM((b_per_w,), jnp.int32),
            pltpu.VMEM((b_per_w, D), jnp.float32),
            [pltpu.SemaphoreType.DMA] * _NCHUNK,
            [pltpu.SemaphoreType.DMA] * _NCHUNK,
        ],
    )
    def gather_kernel(table_hbm, idx_hbm, out_hbm, idx_v, rows_v,
                      gsems, osems):
        wid = lax.axis_index("s")
        base = wid * b_per_w
        pltpu.sync_copy(idx_hbm.at[pl.ds(base, b_per_w)], idx_v)
        gathers = []
        for c in range(_NCHUNK):
            gathers.append(pltpu.async_copy(
                table_hbm.at[idx_v.at[pl.ds(c * cs, cs)]],
                rows_v.at[pl.ds(c * cs, cs)], gsems[c]))
        writes = []
        for c in range(_NCHUNK):
            gathers[c].wait()
            writes.append(pltpu.async_copy(
                rows_v.at[pl.ds(c * cs, cs)],
                out_hbm.at[pl.ds(base + c * cs, cs)], osems[c]))
        for w in writes:
            w.wait()

    return gather_kernel(table, idx)


def _tc_mlp(g, ig, W1, b1, W2, b2, W3, b3):
    B = g.shape[0]
    blk = 8192
    grid = (B // blk,)

    def body(g_ref, ig_ref, w1_ref, b1_ref, w2_ref, b2_ref, w3_ref,
             b3_ref, out_ref):
        bf = jnp.bfloat16
        x = g_ref[...]
        w1 = w1_ref[...]
        emb = x[:, 2:10].astype(bf)
        dn = (((1,), (1,)), ((), ()))
        h = lax.dot_general(emb, w1[:, :8].astype(bf), dn,
                            preferred_element_type=jnp.float32)
        h = h + ig_ref[...] * w1[:, 8][None, :] + b1_ref[...]
        h = jnp.maximum(h, 0.0)
        h = lax.dot_general(h.astype(bf), w2_ref[...].astype(bf), dn,
                            preferred_element_type=jnp.float32)
        h = jnp.maximum(h + b2_ref[...], 0.0)
        res = lax.dot_general(h.astype(bf), w3_ref[...].astype(bf), dn,
                              preferred_element_type=jnp.float32)
        out_ref[...] = x[:, 0:2] + res + b3_ref[...]

    full = lambda shape: pl.BlockSpec(shape, lambda i: (0, 0))
    return pl.pallas_call(
        body,
        grid=grid,
        in_specs=[
            pl.BlockSpec((blk, _TBL_W), lambda i: (i, 0)),
            pl.BlockSpec((blk, 1), lambda i: (i, 0)),
            full((32, 9)),
            full((1, 32)),
            full((16, 32)),
            full((1, 16)),
            full((2, 16)),
            full((1, 2)),
        ],
        out_specs=pl.BlockSpec((blk, 2), lambda i: (i, 0)),
        out_shape=jax.ShapeDtypeStruct((B, 2), jnp.float32),
    )(g, ig, W1, b1, W2, b2, W3, b3)


def kernel(action_idx, is_ground, physics_params, action_emb,
           W1, b1, W2, b2, W3, b3, gravity):
    B = action_idx.shape[0]
    V = physics_params.shape[0]
    idx = action_idx.astype(jnp.int32)
    table = jnp.concatenate(
        [physics_params[:, :2], action_emb,
         jnp.zeros((V, _TBL_W - 10), jnp.float32)], axis=1)
    g = _sc_gather(table, idx)
    out = _tc_mlp(g, is_ground.reshape(B, 1), W1, b1.reshape(1, 32),
                  W2, b2.reshape(1, 16), W3, b3.reshape(1, 2))
    return (out, gravity)
